# Initial kernel scaffold; baseline (speedup 1.0000x reference)
import jax
import jax.numpy as jnp
from jax.experimental import pallas as pl


def kernel(adj_indices, adj_values, uadj_indices, uadj_values, uEmbeds, iEmbeds):
    raise NotImplementedError("write your pallas kernel here")


# speedup vs baseline: 5.1272x; 5.1272x over previous
import functools

import jax
import jax.numpy as jnp
from jax import lax
from jax.experimental import pallas as pl
from jax.experimental.pallas import tpu as pltpu
from jax.experimental.pallas import tpu_sc as plsc

USER_N = 25000
ITEM_N = 25000
NN = USER_N + ITEM_N
D = 64
CB = 32
LANES = 16
CHUNK = 128
JJ = 4
SUP = CHUNK * JJ
NSUB = 16
NCORE = 2
RB = 40


def _build_sc_kernel(k_ui: int, k_uu: int):
    s_ui = k_ui * NSUB
    s_uu = k_uu * NSUB
    ui_blocks = NN // RB
    uu_blocks = USER_N // RB

    mesh = plsc.VectorSubcoreMesh(core_axis_name="c", subcore_axis_name="s")

    @functools.partial(
        pl.kernel,
        out_type=(
            jax.ShapeDtypeStruct((NCORE, NN, CB), jnp.float32),
            jax.ShapeDtypeStruct((NCORE, USER_N, CB), jnp.float32),
            jax.ShapeDtypeStruct((NCORE * NN, CB), jnp.float32),
        ),
        mesh=mesh,
        compiler_params=pltpu.CompilerParams(use_tc_tiling_on_sc=False),
        scratch_types=(
            pltpu.VMEM_SHARED((NN, CB), jnp.float32),
            pltpu.VMEM((JJ, CHUNK), jnp.int32),
            pltpu.VMEM((JJ, CHUNK), jnp.int32),
            pltpu.VMEM((JJ, CHUNK), jnp.float32),
            pltpu.VMEM((JJ, CHUNK, CB), jnp.float32),
            pltpu.VMEM((RB, CB), jnp.float32),
            pltpu.VMEM((RB, CB), jnp.float32),
            pltpu.VMEM((RB, CB), jnp.float32),
            pltpu.VMEM((RB, CB), jnp.float32),
            pltpu.SemaphoreType.DMA,
        ),
    )
    def sc_kernel(ui_dst, ui_src2, ui_val, uu_dst, uu_src2, uu_val, emb,
                  out_ui, out_uu, l1,
                  acc, dbuf, sbuf, vbuf, rows, zbuf, ta, tb, tc, gsem):
        c = lax.axis_index("c")
        s = lax.axis_index("s")
        coff = c * NN

        zeros16 = jnp.zeros((LANES,), jnp.float32)

        def zfill(r, _):
            zbuf[r, pl.ds(0, LANES)] = zeros16
            zbuf[r, pl.ds(LANES, LANES)] = zeros16
            return 0

        lax.fori_loop(0, RB, zfill, 0)

        def zero_acc(nblocks):
            nt = (nblocks + NSUB - 1) // NSUB

            def bd(t, _):
                blk = s + t * NSUB

                @pl.when(blk < nblocks)
                def _():
                    pltpu.sync_copy(zbuf, acc.at[pl.ds(blk * RB, RB)])

                return 0

            lax.fori_loop(0, nt, bd, 0)

        def edge_pass(dst_h, src2_h, val_h, k_tile, table):
            def bd(k, _):
                sup = s * k_tile + k
                pltpu.sync_copy(dst_h.at[sup], dbuf)
                pltpu.sync_copy(src2_h.at[c, sup], sbuf)
                pltpu.sync_copy(val_h.at[sup], vbuf)
                cps = []
                for j in range(JJ):
                    cps.append(
                        pltpu.async_copy(table.at[sbuf.at[j]], rows.at[j], gsem))
                for cp in cps:
                    cp.wait()
                for j in range(JJ):
                    def scale(g, _):
                        val16 = vbuf[j, pl.ds(g * LANES, LANES)]
                        for i in range(LANES):
                            v = val16[i]
                            r = g * LANES + i
                            rows[j, r, pl.ds(0, LANES)] = (
                                rows[j, r, pl.ds(0, LANES)] * v)
                            rows[j, r, pl.ds(LANES, LANES)] = (
                                rows[j, r, pl.ds(LANES, LANES)] * v)
                        return 0

                    lax.fori_loop(0, CHUNK // LANES, scale, 0)
                for j in range(JJ):
                    pltpu.sync_copy(rows.at[j], acc.at[dbuf.at[j]], add=True)
                return 0

            lax.fori_loop(0, k_tile, bd, 0)

        def dump_acc(nblocks):
            nt = (nblocks + NSUB - 1) // NSUB

            def bd(t, _):
                blk = s + t * NSUB

                @pl.when(blk < nblocks)
                def _():
                    r0 = blk * RB
                    pltpu.sync_copy(acc.at[pl.ds(r0, RB)], ta)
                    pltpu.sync_copy(ta, l1.at[pl.ds(coff + r0, RB)])

                return 0

            lax.fori_loop(0, nt, bd, 0)

        def pooled(nblocks, out_ref):
            nt = (nblocks + NSUB - 1) // NSUB

            def bd(t, _):
                blk = s + t * NSUB

                @pl.when(blk < nblocks)
                def _():
                    r0 = blk * RB
                    pltpu.sync_copy(emb.at[pl.ds(coff + r0, RB)], ta)
                    pltpu.sync_copy(l1.at[pl.ds(coff + r0, RB)], tb)
                    pltpu.sync_copy(acc.at[pl.ds(r0, RB)], tc)

                    def add(r, _):
                        ta[r, pl.ds(0, LANES)] = (
                            ta[r, pl.ds(0, LANES)]
                            + tb[r, pl.ds(0, LANES)]
                            + tc[r, pl.ds(0, LANES)])
                        ta[r, pl.ds(LANES, LANES)] = (
                            ta[r, pl.ds(LANES, LANES)]
                            + tb[r, pl.ds(LANES, LANES)]
                            + tc[r, pl.ds(LANES, LANES)])
                        return 0

                    lax.fori_loop(0, RB, add, 0)
                    pltpu.sync_copy(ta, out_ref.at[c, pl.ds(r0, RB)])

                return 0

            lax.fori_loop(0, nt, bd, 0)

        def graph(dst_h, src2_h, val_h, k_tile, nblocks, out_ref):
            zero_acc(nblocks)
            plsc.subcore_barrier()
            edge_pass(dst_h, src2_h, val_h, k_tile, emb)
            plsc.subcore_barrier()
            dump_acc(nblocks)
            plsc.subcore_barrier()
            zero_acc(nblocks)
            plsc.subcore_barrier()
            edge_pass(dst_h, src2_h, val_h, k_tile, l1)
            plsc.subcore_barrier()
            pooled(nblocks, out_ref)
            plsc.subcore_barrier()

        graph(ui_dst, ui_src2, ui_val, k_ui, ui_blocks, out_ui)
        graph(uu_dst, uu_src2, uu_val, k_uu, uu_blocks, out_uu)

    return sc_kernel


def _prep_edges(indices, values, k_tile):
    e = values.shape[0]
    e_pad = k_tile * NSUB * SUP
    pad = e_pad - e
    dst = jnp.concatenate([indices[0], jnp.zeros((pad,), jnp.int32)])
    src = jnp.concatenate([indices[1], jnp.zeros((pad,), jnp.int32)])
    val = jnp.concatenate([values, jnp.zeros((pad,), jnp.float32)])
    dst = dst.reshape(-1, JJ, CHUNK)
    val = val.reshape(-1, JJ, CHUNK)
    src2 = jnp.stack([src, src + NN]).reshape(NCORE, -1, JJ, CHUNK)
    return dst, src2, val


def kernel(adj_indices, adj_values, uadj_indices, uadj_values, uEmbeds, iEmbeds):
    e_ui = adj_values.shape[0]
    e_uu = uadj_values.shape[0]
    k_ui = -(-e_ui // (NSUB * SUP))
    k_uu = -(-e_uu // (NSUB * SUP))

    ui_dst, ui_src2, ui_val = _prep_edges(adj_indices, adj_values, k_ui)
    uu_dst, uu_src2, uu_val = _prep_edges(uadj_indices, uadj_values, k_uu)

    emb = jnp.concatenate([uEmbeds, iEmbeds], axis=0)
    emb_cat = emb.reshape(NN, NCORE, CB).transpose(1, 0, 2).reshape(NCORE * NN, CB)

    sc = _build_sc_kernel(k_ui, k_uu)
    out_ui, out_uu, _ = sc(ui_dst, ui_src2, ui_val, uu_dst, uu_src2, uu_val,
                           emb_cat)

    pooled = jnp.transpose(out_ui, (1, 0, 2)).reshape(NN, D)
    uu = jnp.transpose(out_uu, (1, 0, 2)).reshape(USER_N, D)
    return pooled[:USER_N], pooled[USER_N:], uu


# speedup vs baseline: 6.6148x; 1.2901x over previous
import functools

import jax
import jax.numpy as jnp
from jax import lax
from jax.experimental import pallas as pl
from jax.experimental.pallas import tpu as pltpu
from jax.experimental.pallas import tpu_sc as plsc

USER_N = 25000
ITEM_N = 25000
NN = USER_N + ITEM_N
D = 64
CB = 32
LANES = 16
CHUNK = 128
JJ = 4
SUP = CHUNK * JJ
NSUB = 16
NCORE = 2
RB = 40


def _build_sc_kernel(k_ui: int, k_uu: int):
    s_ui = k_ui * NSUB
    s_uu = k_uu * NSUB
    ui_blocks = NN // RB
    uu_blocks = USER_N // RB

    mesh = plsc.VectorSubcoreMesh(core_axis_name="c", subcore_axis_name="s")

    @functools.partial(
        pl.kernel,
        out_type=(
            jax.ShapeDtypeStruct((NCORE, NN, CB), jnp.float32),
            jax.ShapeDtypeStruct((NCORE, USER_N, CB), jnp.float32),
            jax.ShapeDtypeStruct((NCORE * NN, CB), jnp.float32),
        ),
        mesh=mesh,
        compiler_params=pltpu.CompilerParams(use_tc_tiling_on_sc=False),
        scratch_types=(
            pltpu.VMEM_SHARED((NN, CB), jnp.float32),
            [pltpu.VMEM((JJ, CHUNK), jnp.int32) for _ in range(2)],
            [pltpu.VMEM((JJ, CHUNK), jnp.int32) for _ in range(2)],
            [pltpu.VMEM((JJ, CHUNK), jnp.float32) for _ in range(2)],
            [pltpu.VMEM((CHUNK, CB), jnp.float32) for _ in range(JJ)],
            pltpu.VMEM((RB, CB), jnp.float32),
            pltpu.VMEM((RB, CB), jnp.float32),
            pltpu.VMEM((RB, CB), jnp.float32),
            pltpu.VMEM((RB, CB), jnp.float32),
            [pltpu.SemaphoreType.DMA for _ in range(2)],
            [pltpu.SemaphoreType.DMA for _ in range(JJ)],
            [pltpu.SemaphoreType.DMA for _ in range(2)],
        ),
    )
    def sc_kernel(ui_dst, ui_src2, ui_val, uu_dst, uu_src2, uu_val, emb,
                  out_ui, out_uu, l1,
                  acc, dbufs, sbufs, vbufs, rowss, zbuf, ta, tb, tc,
                  stsems, gsems, ssems):
        c = lax.axis_index("c")
        s = lax.axis_index("s")
        coff = c * NN

        zeros16 = jnp.zeros((LANES,), jnp.float32)

        def zfill(r, _):
            zbuf[r, pl.ds(0, LANES)] = zeros16
            zbuf[r, pl.ds(LANES, LANES)] = zeros16
            return 0

        lax.fori_loop(0, RB, zfill, 0)

        def zero_acc(nblocks):
            nt = (nblocks + NSUB - 1) // NSUB

            def bd(t, _):
                blk = s + t * NSUB

                @pl.when(blk < nblocks)
                def _():
                    pltpu.sync_copy(zbuf, acc.at[pl.ds(blk * RB, RB)])

                return 0

            lax.fori_loop(0, nt, bd, 0)

        def edge_pass(dst_h, src2_h, val_h, k_tile, table):
            def fire_stage(w, q):
                sup = s * k_tile + w
                pltpu.async_copy(dst_h.at[sup], dbufs[q], stsems[q])
                pltpu.async_copy(src2_h.at[c, sup], sbufs[q], stsems[q])
                pltpu.async_copy(val_h.at[sup], vbufs[q], stsems[q])

            def drain_stage(q):
                pltpu.make_async_copy(dst_h.at[0], dbufs[q], stsems[q]).wait()
                pltpu.make_async_copy(src2_h.at[c, 0], sbufs[q], stsems[q]).wait()
                pltpu.make_async_copy(val_h.at[0], vbufs[q], stsems[q]).wait()

            def drain_scatter(q):
                for j in range(JJ):
                    pltpu.make_async_copy(
                        rowss[j], acc.at[dbufs[q].at[j]], ssems[q]).wait()

            def window(w, p):
                @pl.when(w > 0)
                def _():
                    drain_scatter(1 - p)

                drain_stage(p)
                gcps = []
                for j in range(JJ):
                    gcps.append(pltpu.async_copy(
                        table.at[sbufs[p].at[j]], rowss[j], gsems[j]))
                @pl.when(w + 1 < k_tile)
                def _():
                    fire_stage(w + 1, 1 - p)

                for j in range(JJ):
                    gcps[j].wait()

                    def scale(g, _):
                        val16 = vbufs[p][j, pl.ds(g * LANES, LANES)]
                        for i in range(LANES):
                            v = val16[i]
                            r = g * LANES + i
                            rowss[j][r, pl.ds(0, LANES)] = (
                                rowss[j][r, pl.ds(0, LANES)] * v)
                            rowss[j][r, pl.ds(LANES, LANES)] = (
                                rowss[j][r, pl.ds(LANES, LANES)] * v)
                        return 0

                    lax.fori_loop(0, CHUNK // LANES, scale, 0)
                    pltpu.async_copy(
                        rowss[j], acc.at[dbufs[p].at[j]], ssems[p], add=True)

            fire_stage(0, 0)

            def bd(h, _):
                window(2 * h, 0)
                window(2 * h + 1, 1)
                return 0

            lax.fori_loop(0, k_tile // 2, bd, 0)
            drain_scatter((k_tile - 1) % 2)

        def dump_acc(nblocks):
            nt = (nblocks + NSUB - 1) // NSUB

            def bd(t, _):
                blk = s + t * NSUB

                @pl.when(blk < nblocks)
                def _():
                    r0 = blk * RB
                    pltpu.sync_copy(acc.at[pl.ds(r0, RB)], ta)
                    pltpu.sync_copy(ta, l1.at[pl.ds(coff + r0, RB)])

                return 0

            lax.fori_loop(0, nt, bd, 0)

        def pooled(nblocks, out_ref):
            nt = (nblocks + NSUB - 1) // NSUB

            def bd(t, _):
                blk = s + t * NSUB

                @pl.when(blk < nblocks)
                def _():
                    r0 = blk * RB
                    pltpu.sync_copy(emb.at[pl.ds(coff + r0, RB)], ta)
                    pltpu.sync_copy(l1.at[pl.ds(coff + r0, RB)], tb)
                    pltpu.sync_copy(acc.at[pl.ds(r0, RB)], tc)

                    def add(r, _):
                        ta[r, pl.ds(0, LANES)] = (
                            ta[r, pl.ds(0, LANES)]
                            + tb[r, pl.ds(0, LANES)]
                            + tc[r, pl.ds(0, LANES)])
                        ta[r, pl.ds(LANES, LANES)] = (
                            ta[r, pl.ds(LANES, LANES)]
                            + tb[r, pl.ds(LANES, LANES)]
                            + tc[r, pl.ds(LANES, LANES)])
                        return 0

                    lax.fori_loop(0, RB, add, 0)
                    pltpu.sync_copy(ta, out_ref.at[c, pl.ds(r0, RB)])

                return 0

            lax.fori_loop(0, nt, bd, 0)

        def graph(dst_h, src2_h, val_h, k_tile, nblocks, out_ref):
            zero_acc(nblocks)
            plsc.subcore_barrier()
            edge_pass(dst_h, src2_h, val_h, k_tile, emb)
            plsc.subcore_barrier()
            dump_acc(nblocks)
            plsc.subcore_barrier()
            zero_acc(nblocks)
            plsc.subcore_barrier()
            edge_pass(dst_h, src2_h, val_h, k_tile, l1)
            plsc.subcore_barrier()
            pooled(nblocks, out_ref)
            plsc.subcore_barrier()

        graph(ui_dst, ui_src2, ui_val, k_ui, ui_blocks, out_ui)
        graph(uu_dst, uu_src2, uu_val, k_uu, uu_blocks, out_uu)

    return sc_kernel


def _prep_edges(indices, values, k_tile):
    e = values.shape[0]
    e_pad = k_tile * NSUB * SUP
    pad = e_pad - e
    dst = jnp.concatenate([indices[0], jnp.zeros((pad,), jnp.int32)])
    src = jnp.concatenate([indices[1], jnp.zeros((pad,), jnp.int32)])
    val = jnp.concatenate([values, jnp.zeros((pad,), jnp.float32)])
    dst = dst.reshape(-1, JJ, CHUNK)
    val = val.reshape(-1, JJ, CHUNK)
    src2 = jnp.stack([src, src + NN]).reshape(NCORE, -1, JJ, CHUNK)
    return dst, src2, val


def kernel(adj_indices, adj_values, uadj_indices, uadj_values, uEmbeds, iEmbeds):
    e_ui = adj_values.shape[0]
    e_uu = uadj_values.shape[0]
    k_ui = -(-e_ui // (NSUB * SUP))
    k_uu = -(-e_uu // (NSUB * SUP))
    k_ui += k_ui % 2
    k_uu += k_uu % 2

    ui_dst, ui_src2, ui_val = _prep_edges(adj_indices, adj_values, k_ui)
    uu_dst, uu_src2, uu_val = _prep_edges(uadj_indices, uadj_values, k_uu)

    emb = jnp.concatenate([uEmbeds, iEmbeds], axis=0)
    emb_cat = emb.reshape(NN, NCORE, CB).transpose(1, 0, 2).reshape(NCORE * NN, CB)

    sc = _build_sc_kernel(k_ui, k_uu)
    out_ui, out_uu, _ = sc(ui_dst, ui_src2, ui_val, uu_dst, uu_src2, uu_val,
                           emb_cat)

    pooled = jnp.transpose(out_ui, (1, 0, 2)).reshape(NN, D)
    uu = jnp.transpose(out_uu, (1, 0, 2)).reshape(USER_N, D)
    return pooled[:USER_N], pooled[USER_N:], uu


# speedup vs baseline: 7.0960x; 1.0727x over previous
import functools

import jax
import jax.numpy as jnp
from jax import lax
from jax.experimental import pallas as pl
from jax.experimental.pallas import tpu as pltpu
from jax.experimental.pallas import tpu_sc as plsc

USER_N = 25000
ITEM_N = 25000
NN = USER_N + ITEM_N
D = 64
CB = 32
LANES = 16
CHUNK = 64
JJ = 4
SUP = CHUNK * JJ
NSUB = 16
NCORE = 2
RB = 40


def _build_sc_kernel(k_ui: int, k_uu: int):
    ui_blocks = NN // RB
    uu_blocks = USER_N // RB

    mesh = plsc.VectorSubcoreMesh(core_axis_name="c", subcore_axis_name="s")

    @functools.partial(
        pl.kernel,
        out_type=(
            jax.ShapeDtypeStruct((NCORE, NN, CB), jnp.float32),
            jax.ShapeDtypeStruct((NCORE, USER_N, CB), jnp.float32),
            jax.ShapeDtypeStruct((NCORE * NN, CB), jnp.float32),
        ),
        mesh=mesh,
        compiler_params=pltpu.CompilerParams(use_tc_tiling_on_sc=False),
        scratch_types=(
            pltpu.VMEM_SHARED((NN, CB), jnp.float32),
            [pltpu.VMEM((JJ, CHUNK), jnp.int32) for _ in range(2)],
            [pltpu.VMEM((JJ, CHUNK), jnp.int32) for _ in range(2)],
            [pltpu.VMEM((JJ, CHUNK), jnp.float32) for _ in range(2)],
            [[pltpu.VMEM((CHUNK, CB), jnp.float32) for _ in range(JJ)]
             for _ in range(2)],
            pltpu.VMEM((RB, CB), jnp.float32),
            pltpu.VMEM((RB, CB), jnp.float32),
            pltpu.VMEM((RB, CB), jnp.float32),
            pltpu.VMEM((RB, CB), jnp.float32),
            [pltpu.SemaphoreType.DMA for _ in range(2)],
            [pltpu.SemaphoreType.DMA for _ in range(JJ)],
            [pltpu.SemaphoreType.DMA for _ in range(2)],
        ),
    )
    def sc_kernel(ui_dst, ui_src2, ui_val, uu_dst, uu_src2, uu_val, emb,
                  out_ui, out_uu, l1,
                  acc, dbufs, sbufs, vbufs, rowss, zbuf, ta, tb, tc,
                  stsems, gsems, ssems):
        c = lax.axis_index("c")
        s = lax.axis_index("s")
        coff = c * NN

        zeros16 = jnp.zeros((LANES,), jnp.float32)

        def zfill(r, _):
            zbuf[r, pl.ds(0, LANES)] = zeros16
            zbuf[r, pl.ds(LANES, LANES)] = zeros16
            return 0

        lax.fori_loop(0, RB, zfill, 0)

        def zero_acc(nblocks):
            nt = (nblocks + NSUB - 1) // NSUB

            def bd(t, _):
                blk = s + t * NSUB

                @pl.when(blk < nblocks)
                def _():
                    pltpu.sync_copy(zbuf, acc.at[pl.ds(blk * RB, RB)])

                return 0

            lax.fori_loop(0, nt, bd, 0)

        def edge_pass(dst_h, src2_h, val_h, k_tile, table):
            def fire_stage(w, q):
                sup = s * k_tile + w
                pltpu.async_copy(dst_h.at[sup], dbufs[q], stsems[q])
                pltpu.async_copy(src2_h.at[c, sup], sbufs[q], stsems[q])
                pltpu.async_copy(val_h.at[sup], vbufs[q], stsems[q])

            def drain_stage(q):
                pltpu.make_async_copy(dst_h.at[0], dbufs[q], stsems[q]).wait()
                pltpu.make_async_copy(src2_h.at[c, 0], sbufs[q], stsems[q]).wait()
                pltpu.make_async_copy(val_h.at[0], vbufs[q], stsems[q]).wait()

            def drain_scatter(q):
                for j in range(JJ):
                    pltpu.make_async_copy(
                        rowss[q][j], acc.at[dbufs[q].at[j]], ssems[q]).wait()

            def window(w, p):
                drain_stage(p)
                gcps = []
                for j in range(JJ):
                    gcps.append(pltpu.async_copy(
                        table.at[sbufs[p].at[j]], rowss[p][j], gsems[j]))

                @pl.when(w > 0)
                def _():
                    drain_scatter(1 - p)

                @pl.when(w + 1 < k_tile)
                def _():
                    fire_stage(w + 1, 1 - p)

                for j in range(JJ):
                    gcps[j].wait()

                    def scale(g, _):
                        val16 = vbufs[p][j, pl.ds(g * LANES, LANES)]
                        for i in range(LANES):
                            v = val16[i]
                            r = g * LANES + i
                            rowss[p][j][r, pl.ds(0, LANES)] = (
                                rowss[p][j][r, pl.ds(0, LANES)] * v)
                            rowss[p][j][r, pl.ds(LANES, LANES)] = (
                                rowss[p][j][r, pl.ds(LANES, LANES)] * v)
                        return 0

                    lax.fori_loop(0, CHUNK // LANES, scale, 0)
                    pltpu.async_copy(
                        rowss[p][j], acc.at[dbufs[p].at[j]], ssems[p],
                        add=True)

            fire_stage(0, 0)

            def bd(h, _):
                window(2 * h, 0)
                window(2 * h + 1, 1)
                return 0

            lax.fori_loop(0, k_tile // 2, bd, 0)
            drain_scatter((k_tile - 1) % 2)

        def dump_acc(nblocks):
            nt = (nblocks + NSUB - 1) // NSUB

            def bd(t, _):
                blk = s + t * NSUB

                @pl.when(blk < nblocks)
                def _():
                    r0 = blk * RB
                    pltpu.sync_copy(acc.at[pl.ds(r0, RB)], ta)
                    pltpu.sync_copy(ta, l1.at[pl.ds(coff + r0, RB)])

                return 0

            lax.fori_loop(0, nt, bd, 0)

        def pooled(nblocks, out_ref):
            nt = (nblocks + NSUB - 1) // NSUB

            def bd(t, _):
                blk = s + t * NSUB

                @pl.when(blk < nblocks)
                def _():
                    r0 = blk * RB
                    pltpu.sync_copy(emb.at[pl.ds(coff + r0, RB)], ta)
                    pltpu.sync_copy(l1.at[pl.ds(coff + r0, RB)], tb)
                    pltpu.sync_copy(acc.at[pl.ds(r0, RB)], tc)

                    def add(r, _):
                        ta[r, pl.ds(0, LANES)] = (
                            ta[r, pl.ds(0, LANES)]
                            + tb[r, pl.ds(0, LANES)]
                            + tc[r, pl.ds(0, LANES)])
                        ta[r, pl.ds(LANES, LANES)] = (
                            ta[r, pl.ds(LANES, LANES)]
                            + tb[r, pl.ds(LANES, LANES)]
                            + tc[r, pl.ds(LANES, LANES)])
                        return 0

                    lax.fori_loop(0, RB, add, 0)
                    pltpu.sync_copy(ta, out_ref.at[c, pl.ds(r0, RB)])

                return 0

            lax.fori_loop(0, nt, bd, 0)

        def graph(dst_h, src2_h, val_h, k_tile, nblocks, out_ref):
            zero_acc(nblocks)
            plsc.subcore_barrier()
            edge_pass(dst_h, src2_h, val_h, k_tile, emb)
            plsc.subcore_barrier()
            dump_acc(nblocks)
            plsc.subcore_barrier()
            zero_acc(nblocks)
            plsc.subcore_barrier()
            edge_pass(dst_h, src2_h, val_h, k_tile, l1)
            plsc.subcore_barrier()
            pooled(nblocks, out_ref)
            plsc.subcore_barrier()

        graph(ui_dst, ui_src2, ui_val, k_ui, ui_blocks, out_ui)
        graph(uu_dst, uu_src2, uu_val, k_uu, uu_blocks, out_uu)

    return sc_kernel


def _prep_edges(indices, values, k_tile):
    e = values.shape[0]
    e_pad = k_tile * NSUB * SUP
    pad = e_pad - e
    dst = jnp.concatenate([indices[0], jnp.zeros((pad,), jnp.int32)])
    src = jnp.concatenate([indices[1], jnp.zeros((pad,), jnp.int32)])
    val = jnp.concatenate([values, jnp.zeros((pad,), jnp.float32)])
    dst = dst.reshape(-1, JJ, CHUNK)
    val = val.reshape(-1, JJ, CHUNK)
    src2 = jnp.stack([src, src + NN]).reshape(NCORE, -1, JJ, CHUNK)
    return dst, src2, val


def kernel(adj_indices, adj_values, uadj_indices, uadj_values, uEmbeds, iEmbeds):
    e_ui = adj_values.shape[0]
    e_uu = uadj_values.shape[0]
    k_ui = -(-e_ui // (NSUB * SUP))
    k_uu = -(-e_uu // (NSUB * SUP))
    k_ui += k_ui % 2
    k_uu += k_uu % 2

    ui_dst, ui_src2, ui_val = _prep_edges(adj_indices, adj_values, k_ui)
    uu_dst, uu_src2, uu_val = _prep_edges(uadj_indices, uadj_values, k_uu)

    emb = jnp.concatenate([uEmbeds, iEmbeds], axis=0)
    emb_cat = emb.reshape(NN, NCORE, CB).transpose(1, 0, 2).reshape(NCORE * NN, CB)

    sc = _build_sc_kernel(k_ui, k_uu)
    out_ui, out_uu, _ = sc(ui_dst, ui_src2, ui_val, uu_dst, uu_src2, uu_val,
                           emb_cat)

    pooled = jnp.transpose(out_ui, (1, 0, 2)).reshape(NN, D)
    uu = jnp.transpose(out_uu, (1, 0, 2)).reshape(USER_N, D)
    return pooled[:USER_N], pooled[USER_N:], uu
